# Initial kernel scaffold; baseline (speedup 1.0000x reference)
import jax
import jax.numpy as jnp
from jax.experimental import pallas as pl


def kernel(label):
    raise NotImplementedError("write your pallas kernel here")


# speedup vs baseline: 72.8987x; 72.8987x over previous
import functools

import jax
import jax.numpy as jnp
from jax import lax
from jax.experimental import pallas as pl
from jax.experimental.pallas import tpu as pltpu
from jax.experimental.pallas import tpu_sc as plsc

H = 512
W = 512
N = H * W
NC = 2
NS = 16
L = 16
READ = N // NS
ROWS_PER_W = H // (NC * NS)
OUT = ROWS_PER_W * W
THRESH = 11

_mesh = plsc.VectorSubcoreMesh(core_axis_name="c", subcore_axis_name="s")


@functools.partial(
    pl.kernel,
    mesh=_mesh,
    out_type=jax.ShapeDtypeStruct((N,), jnp.float32),
    scratch_types=[
        pltpu.VMEM((READ,), jnp.int32),
        pltpu.VMEM((OUT,), jnp.float32),
        pltpu.VMEM((L,), jnp.int32),
        pltpu.VMEM((NS * L,), jnp.int32),
        pltpu.VMEM_SHARED((NS * L,), jnp.int32),
    ],
)
def _sampler_sc(label_hbm, out_hbm, chunk_v, out_v, flag_v, part_v, shared):
    c = lax.axis_index("c")
    s = lax.axis_index("s")

    pltpu.sync_copy(label_hbm.at[pl.ds(s * READ, READ)], chunk_v)

    def red(i, acc):
        return jnp.maximum(acc, chunk_v[pl.ds(i * L, L)])

    acc = lax.fori_loop(0, READ // L, red, jnp.zeros((L,), jnp.int32))
    flag_v[...] = acc

    pltpu.sync_copy(flag_v, shared.at[pl.ds(s * L, L)])
    plsc.subcore_barrier()
    pltpu.sync_copy(shared, part_v)

    def red2(j, acc):
        return jnp.maximum(acc, part_v[pl.ds(j * L, L)])

    acc2 = lax.fori_loop(0, NS, red2, jnp.zeros((L,), jnp.int32))
    gmax = acc2[0]
    for i in range(1, L):
        gmax = jnp.maximum(gmax, acc2[i])
    val = jnp.where(gmax >= THRESH, 0.0, 1.0).astype(jnp.float32)
    vec = jnp.full((L,), val, jnp.float32)

    def fill(i, carry):
        out_v[pl.ds(i * L, L)] = vec
        return carry

    lax.fori_loop(0, OUT // L, fill, 0)

    @pl.when(jnp.logical_and(c == 0, s == 0))
    def _():
        ones = jnp.full((L,), 1.0, jnp.float32)

        def fill1(i, carry):
            out_v[pl.ds(i * L, L)] = ones
            return carry

        lax.fori_loop(0, (2 * W) // L, fill1, 0)

    base = (c * NS + s) * OUT
    pltpu.sync_copy(out_v, out_hbm.at[pl.ds(base, OUT)])


def kernel(label):
    out = _sampler_sc(label.reshape(-1))
    return out.reshape(H, W)


# speedup vs baseline: 88.6284x; 1.2158x over previous
import functools

import jax
import jax.numpy as jnp
from jax import lax
from jax.experimental import pallas as pl
from jax.experimental.pallas import tpu as pltpu
from jax.experimental.pallas import tpu_sc as plsc

H = 512
W = 512
N = H * W
NC = 2
NS = 16
L = 16
READ = N // NS
ROWS_PER_W = H // (NC * NS)
OUT = ROWS_PER_W * W
THRESH = 11

_mesh = plsc.VectorSubcoreMesh(core_axis_name="c", subcore_axis_name="s")


@functools.partial(
    pl.kernel,
    mesh=_mesh,
    out_type=jax.ShapeDtypeStruct((N,), jnp.float32),
    scratch_types=[
        pltpu.VMEM((READ,), jnp.int32),
        pltpu.VMEM((OUT,), jnp.float32),
        pltpu.VMEM((L,), jnp.int32),
        pltpu.VMEM((NS * L,), jnp.int32),
        pltpu.VMEM_SHARED((NS * L,), jnp.int32),
    ],
)
def _sampler_sc(label_hbm, out_hbm, chunk_v, out_v, flag_v, part_v, shared):
    c = lax.axis_index("c")
    s = lax.axis_index("s")

    pltpu.sync_copy(label_hbm.at[pl.ds(s * READ, READ)], chunk_v)

    U = 8

    @plsc.parallel_loop(0, READ // L, step=U,
                        carry=jnp.zeros((L,), jnp.int32))
    def acc(i, m):
        v = [chunk_v[pl.ds((i + u) * L, L)] for u in range(U)]
        for stride in (4, 2, 1):
            v = [jnp.maximum(v[k], v[k + stride]) for k in range(stride)]
        return jnp.maximum(m, v[0])

    flag_v[...] = acc

    pltpu.sync_copy(flag_v, shared.at[pl.ds(s * L, L)])
    plsc.subcore_barrier()
    pltpu.sync_copy(shared, part_v)

    v2 = [part_v[pl.ds(j * L, L)] for j in range(NS)]
    for stride in (8, 4, 2, 1):
        v2 = [jnp.maximum(v2[k], v2[k + stride]) for k in range(stride)]
    acc2 = v2[0]
    gmax = acc2[0]
    for i in range(1, L):
        gmax = jnp.maximum(gmax, acc2[i])
    val = jnp.where(gmax >= THRESH, 0.0, 1.0).astype(jnp.float32)
    vec = jnp.full((L,), val, jnp.float32)

    @plsc.parallel_loop(0, OUT // L, step=U)
    def _fill(i):
        for u in range(U):
            out_v[pl.ds((i + u) * L, L)] = vec

    @pl.when(jnp.logical_and(c == 0, s == 0))
    def _():
        ones = jnp.full((L,), 1.0, jnp.float32)
        for i in range((2 * W) // L):
            out_v[pl.ds(i * L, L)] = ones

    base = (c * NS + s) * OUT
    pltpu.sync_copy(out_v, out_hbm.at[pl.ds(base, OUT)])


def kernel(label):
    out = _sampler_sc(label.reshape(-1))
    return out.reshape(H, W)


# speedup vs baseline: 97.2569x; 1.0974x over previous
import functools

import jax
import jax.numpy as jnp
from jax import lax
from jax.experimental import pallas as pl
from jax.experimental.pallas import tpu as pltpu
from jax.experimental.pallas import tpu_sc as plsc

H = 512
W = 512
NC = 2
NS = 16
L = 16
RROWS = H // NS
OROWS = H // (NC * NS)
WVECS = W // L

_mesh = plsc.VectorSubcoreMesh(core_axis_name="c", subcore_axis_name="s")


@functools.partial(
    pl.kernel,
    mesh=_mesh,
    out_type=jax.ShapeDtypeStruct((H, W), jnp.float32),
    scratch_types=[
        pltpu.VMEM((RROWS, W), jnp.int32),
        pltpu.VMEM((OROWS, W), jnp.float32),
        pltpu.VMEM((L,), jnp.int32),
        pltpu.VMEM((NS * L,), jnp.int32),
        pltpu.VMEM_SHARED((NS * L,), jnp.int32),
    ],
)
def _sampler_sc(label_hbm, out_hbm, chunk_v, out_v, flag_v, part_v, shared):
    c = lax.axis_index("c")
    s = lax.axis_index("s")

    pltpu.sync_copy(label_hbm.at[pl.ds(s * RROWS, RROWS), :], chunk_v)

    @plsc.parallel_loop(0, WVECS, carry=jnp.zeros((L,), jnp.int32))
    def acc(j, m):
        v = [chunk_v[r, pl.ds(j * L, L)] for r in range(RROWS)]
        stride = RROWS // 2
        while stride >= 1:
            v = [jnp.maximum(v[k], v[k + stride]) for k in range(stride)]
            stride //= 2
        return jnp.maximum(m, v[0])

    flag_v[...] = acc

    pltpu.sync_copy(flag_v, shared.at[pl.ds(s * L, L)])
    plsc.subcore_barrier()
    pltpu.sync_copy(shared, part_v)

    v2 = [part_v[pl.ds(j * L, L)] for j in range(NS)]
    stride = NS // 2
    while stride >= 1:
        v2 = [jnp.maximum(v2[k], v2[k + stride]) for k in range(stride)]
        stride //= 2
    acc2 = v2[0]

    gmax = acc2[0]
    for i in range(1, L):
        gmax = jnp.maximum(gmax, acc2[i])

    val = jnp.where(gmax >= 11, 0.0, 1.0).astype(jnp.float32)
    vec = jnp.full((L,), val, jnp.float32)

    @plsc.parallel_loop(0, OROWS)
    def _fill(r):
        for j in range(WVECS):
            out_v[r, pl.ds(j * L, L)] = vec

    @pl.when(jnp.logical_and(c == 0, s == 0))
    def _():
        ones = jnp.full((L,), 1.0, jnp.float32)
        for r in range(2):
            for j in range(WVECS):
                out_v[r, pl.ds(j * L, L)] = ones

    base = (c * NS + s) * OROWS
    pltpu.sync_copy(out_v, out_hbm.at[pl.ds(base, OROWS), :])


def kernel(label):
    return _sampler_sc(label)


# speedup vs baseline: 101.8516x; 1.0472x over previous
import functools

import jax
import jax.numpy as jnp
from jax import lax
from jax.experimental import pallas as pl
from jax.experimental.pallas import tpu as pltpu
from jax.experimental.pallas import tpu_sc as plsc

H = 512
W = 512
NS = 16
L = 16
RROWS = H // NS
WVECS = W // L

_mesh = plsc.VectorSubcoreMesh(
    core_axis_name="c", subcore_axis_name="s", num_cores=1)


@functools.partial(
    pl.kernel,
    mesh=_mesh,
    out_type=jax.ShapeDtypeStruct((H, W), jnp.float32),
    scratch_types=[
        pltpu.VMEM((RROWS, W), jnp.int32),
        pltpu.VMEM((RROWS, W), jnp.float32),
        pltpu.VMEM((L,), jnp.int32),
        pltpu.VMEM((NS * L,), jnp.int32),
        pltpu.VMEM_SHARED((NS * L,), jnp.int32),
    ],
)
def _sampler_sc(label_hbm, out_hbm, chunk_v, out_v, flag_v, part_v, shared):
    s = lax.axis_index("s")

    pltpu.sync_copy(label_hbm.at[pl.ds(s * RROWS, RROWS), :], chunk_v)

    @plsc.parallel_loop(0, WVECS, carry=jnp.zeros((L,), jnp.int32))
    def acc(j, m):
        v = [chunk_v[r, pl.ds(j * L, L)] for r in range(RROWS)]
        stride = RROWS // 2
        while stride >= 1:
            v = [jnp.maximum(v[k], v[k + stride]) for k in range(stride)]
            stride //= 2
        return jnp.maximum(m, v[0])

    flag_v[...] = acc

    pltpu.sync_copy(flag_v, shared.at[pl.ds(s * L, L)])
    plsc.subcore_barrier()
    pltpu.sync_copy(shared, part_v)

    v2 = [part_v[pl.ds(j * L, L)] for j in range(NS)]
    stride = NS // 2
    while stride >= 1:
        v2 = [jnp.maximum(v2[k], v2[k + stride]) for k in range(stride)]
        stride //= 2
    acc2 = v2[0]

    gmax = acc2[0]
    for i in range(1, L):
        gmax = jnp.maximum(gmax, acc2[i])

    val = jnp.where(gmax >= 11, 0.0, 1.0).astype(jnp.float32)
    vec = jnp.full((L,), val, jnp.float32)

    @plsc.parallel_loop(0, RROWS)
    def _fill(r):
        for j in range(WVECS):
            out_v[r, pl.ds(j * L, L)] = vec

    @pl.when(s == 0)
    def _():
        ones = jnp.full((L,), 1.0, jnp.float32)
        for r in range(2):
            for j in range(WVECS):
                out_v[r, pl.ds(j * L, L)] = ones

    pltpu.sync_copy(out_v, out_hbm.at[pl.ds(s * RROWS, RROWS), :])


def kernel(label):
    return _sampler_sc(label)


# speedup vs baseline: 105.4634x; 1.0355x over previous
import functools

import jax
import jax.numpy as jnp
from jax import lax
from jax.experimental import pallas as pl
from jax.experimental.pallas import tpu as pltpu
from jax.experimental.pallas import tpu_sc as plsc

H = 512
W = 512
NS = 16
L = 16
RROWS = H // NS
HALF = RROWS // 2
WVECS = W // L
PAIRS = RROWS // 2

_mesh = plsc.VectorSubcoreMesh(
    core_axis_name="c", subcore_axis_name="s", num_cores=1)


def _band_max(ref, m):
    @plsc.parallel_loop(0, WVECS, carry=m)
    def acc(j, mm):
        v = [ref[r, pl.ds(j * L, L)] for r in range(HALF)]
        stride = HALF // 2
        while stride >= 1:
            v = [jnp.maximum(v[k], v[k + stride]) for k in range(stride)]
            stride //= 2
        return jnp.maximum(mm, v[0])

    return acc


@functools.partial(
    pl.kernel,
    mesh=_mesh,
    out_type=jax.ShapeDtypeStruct((H, W), jnp.float32),
    scratch_types=[
        pltpu.VMEM((HALF, W), jnp.int32),
        pltpu.VMEM((HALF, W), jnp.int32),
        pltpu.VMEM((2, W), jnp.float32),
        pltpu.VMEM((2, W), jnp.float32),
        pltpu.VMEM((L,), jnp.int32),
        pltpu.VMEM((NS * L,), jnp.int32),
        pltpu.VMEM_SHARED((NS * L,), jnp.int32),
        pltpu.SemaphoreType.DMA,
        pltpu.SemaphoreType.DMA,
        pltpu.SemaphoreType.DMA,
    ],
)
def _sampler_sc(label_hbm, out_hbm, chunk_a, chunk_b, val_v, ones_v,
                flag_v, part_v, shared, sem_a, sem_b, sem_o):
    s = lax.axis_index("s")
    row0 = s * RROWS

    cp_a = pltpu.async_copy(label_hbm.at[pl.ds(row0, HALF), :], chunk_a,
                            sem_a)
    cp_b = pltpu.async_copy(label_hbm.at[pl.ds(row0 + HALF, HALF), :],
                            chunk_b, sem_b)
    cp_a.wait()
    m = _band_max(chunk_a, jnp.zeros((L,), jnp.int32))
    cp_b.wait()
    m = _band_max(chunk_b, m)
    flag_v[...] = m

    pltpu.sync_copy(flag_v, shared.at[pl.ds(s * L, L)])
    plsc.subcore_barrier()
    pltpu.sync_copy(shared, part_v)

    v2 = [part_v[pl.ds(j * L, L)] for j in range(NS)]
    stride = NS // 2
    while stride >= 1:
        v2 = [jnp.maximum(v2[k], v2[k + stride]) for k in range(stride)]
        stride //= 2
    acc2 = v2[0]

    gmax = acc2[0]
    for i in range(1, L):
        gmax = jnp.maximum(gmax, acc2[i])

    val = jnp.where(gmax >= 11, 0.0, 1.0).astype(jnp.float32)
    vec = jnp.full((L,), val, jnp.float32)
    ones = jnp.full((L,), 1.0, jnp.float32)

    for r in range(2):
        for j in range(WVECS):
            val_v[r, pl.ds(j * L, L)] = vec
            ones_v[r, pl.ds(j * L, L)] = ones

    copies = [
        pltpu.async_copy(val_v, out_hbm.at[pl.ds(row0 + 2 * p, 2), :],
                         sem_o)
        for p in range(PAIRS)
    ]
    for cp in copies:
        cp.wait()

    @pl.when(s == 0)
    def _():
        pltpu.sync_copy(ones_v, out_hbm.at[pl.ds(0, 2), :])


def kernel(label):
    return _sampler_sc(label)
